# baseline (speedup 1.0000x reference)
import jax
import jax.numpy as jnp
from jax.experimental import pallas as pl
from jax.experimental.pallas import tpu as pltpu

MAX_H = 64
MAX_W = 64

CH = 2
NSLOT = 8


def kernel(x, pos_table):
    B, H, W, C = x.shape
    pt_r = pos_table.reshape(MAX_H, MAX_W, C)
    nchunk = B // CH

    def body(x_hbm, pt_ref, o_hbm, in_buf, out_buf, recv_sems, send_sems):
        def start_recv(chunk, slot):
            pltpu.make_async_copy(
                x_hbm.at[pl.ds(chunk * CH, CH)], in_buf.at[slot], recv_sems.at[slot]
            ).start()

        for s in range(NSLOT):
            start_recv(s, s)

        def step(i, carry):
            slot = jax.lax.rem(i, NSLOT)
            pltpu.make_async_copy(
                x_hbm.at[pl.ds(i * CH, CH)], in_buf.at[slot], recv_sems.at[slot]
            ).wait()

            @pl.when(i >= NSLOT)
            def _():
                pltpu.make_async_copy(
                    out_buf.at[slot],
                    o_hbm.at[pl.ds((i - NSLOT) * CH, CH)],
                    send_sems.at[slot],
                ).wait()

            out_buf[slot] = in_buf[slot] + pt_ref[:H, :W, :][None]

            pltpu.make_async_copy(
                out_buf.at[slot], o_hbm.at[pl.ds(i * CH, CH)], send_sems.at[slot]
            ).start()

            @pl.when(i + NSLOT < nchunk)
            def _():
                start_recv(i + NSLOT, slot)

            return carry

        jax.lax.fori_loop(0, nchunk, step, 0)

        for s in range(NSLOT):
            pltpu.make_async_copy(
                out_buf.at[s], o_hbm.at[pl.ds(0, CH)], send_sems.at[s]
            ).wait()

    return pl.pallas_call(
        body,
        in_specs=[
            pl.BlockSpec(memory_space=pl.ANY),
            pl.BlockSpec(memory_space=pltpu.MemorySpace.VMEM),
        ],
        out_specs=pl.BlockSpec(memory_space=pl.ANY),
        out_shape=jax.ShapeDtypeStruct((B, H, W, C), x.dtype),
        scratch_shapes=[
            pltpu.VMEM((NSLOT, CH, H, W, C), x.dtype),
            pltpu.VMEM((NSLOT, CH, H, W, C), x.dtype),
            pltpu.SemaphoreType.DMA((NSLOT,)),
            pltpu.SemaphoreType.DMA((NSLOT,)),
        ],
    )(x, pt_r)


# speedup vs baseline: 1.0037x; 1.0037x over previous
import jax
import jax.numpy as jnp
from jax.experimental import pallas as pl
from jax.experimental.pallas import tpu as pltpu

MAX_H = 64
MAX_W = 64

CH = 2
NSLOT = 8
NPRIO = 2


def kernel(x, pos_table):
    B, H, W, C = x.shape
    pt_r = pos_table.reshape(MAX_H, MAX_W, C)
    nchunk = B // CH
    rounds = nchunk // NSLOT

    def body(x_hbm, pt_ref, o_hbm, in_buf, out_buf, recv_sems, send_sems):
        def start_recv(chunk, slot):
            pltpu.make_async_copy(
                x_hbm.at[pl.ds(chunk * CH, CH)], in_buf.at[slot], recv_sems.at[slot]
            ).start(priority=slot % NPRIO)

        for s in range(NSLOT):
            start_recv(s, s)

        def round_body(r, carry):
            for s in range(NSLOT):
                i = r * NSLOT + s
                pltpu.make_async_copy(
                    x_hbm.at[pl.ds(i * CH, CH)], in_buf.at[s], recv_sems.at[s]
                ).wait()

                @pl.when(r >= 1)
                def _():
                    pltpu.make_async_copy(
                        out_buf.at[s],
                        o_hbm.at[pl.ds((i - NSLOT) * CH, CH)],
                        send_sems.at[s],
                    ).wait()

                out_buf[s] = in_buf[s] + pt_ref[:H, :W, :][None]

                pltpu.make_async_copy(
                    out_buf.at[s], o_hbm.at[pl.ds(i * CH, CH)], send_sems.at[s]
                ).start(priority=s % NPRIO)

                @pl.when(i + NSLOT < nchunk)
                def _():
                    start_recv(i + NSLOT, s)

            return carry

        jax.lax.fori_loop(0, rounds, round_body, 0)

        for s in range(NSLOT):
            pltpu.make_async_copy(
                out_buf.at[s], o_hbm.at[pl.ds(0, CH)], send_sems.at[s]
            ).wait()

    return pl.pallas_call(
        body,
        in_specs=[
            pl.BlockSpec(memory_space=pl.ANY),
            pl.BlockSpec(memory_space=pltpu.MemorySpace.VMEM),
        ],
        out_specs=pl.BlockSpec(memory_space=pl.ANY),
        out_shape=jax.ShapeDtypeStruct((B, H, W, C), x.dtype),
        scratch_shapes=[
            pltpu.VMEM((NSLOT, CH, H, W, C), x.dtype),
            pltpu.VMEM((NSLOT, CH, H, W, C), x.dtype),
            pltpu.SemaphoreType.DMA((NSLOT,)),
            pltpu.SemaphoreType.DMA((NSLOT,)),
        ],
    )(x, pt_r)
